# baseline (speedup 1.0000x reference)
import jax, jax.numpy as jnp
from jax.experimental import pallas as pl

N, E, D, H, C, DE = 10000, 320000, 128, 8, 128, 16


def _final_body(a_r, b_r, c_r, o_r):
    o_r[...] = jax.nn.relu(a_r[...] + b_r[...]) + c_r[...]


def kernel(x, edge_index, edge_attr, Wq, bq, Wk, bk, Wv, bv, We, Wskip, bskip, Wres):
    src = edge_index[0]
    dst = edge_index[1]
    q = (x @ Wq + bq).reshape(N, H, C)
    k = (x @ Wk + bk).reshape(N, H, C)
    v = (x @ Wv + bv).reshape(N, H, C)
    e = (edge_attr @ We).reshape(E, H, C)
    k_j = k[src] + e
    q_i = q[dst]
    logits = (q_i * k_j).sum(-1) / jnp.sqrt(float(C))
    seg_max = jax.ops.segment_max(logits, dst, num_segments=N)
    seg_max = jnp.where(jnp.isfinite(seg_max), seg_max, 0.0)
    ex = jnp.exp(logits - seg_max[dst])
    seg_sum = jax.ops.segment_sum(ex, dst, num_segments=N)
    alpha = ex / (seg_sum[dst] + 1e-16)
    msg = alpha[..., None] * (v[src] + e)
    out = jax.ops.segment_sum(msg, dst, num_segments=N)
    out = out.mean(axis=1)
    skip = x @ Wskip + bskip
    res = x @ Wres
    return pl.pallas_call(
        _final_body,
        grid=(10,),
        in_specs=[pl.BlockSpec((1000, C), lambda i: (i, 0))] * 3,
        out_specs=pl.BlockSpec((1000, C), lambda i: (i, 0)),
        out_shape=jax.ShapeDtypeStruct((N, C), jnp.float32),
    )(out, skip, res)


# speedup vs baseline: 17.7687x; 17.7687x over previous
import functools

import jax
import jax.numpy as jnp
from jax import lax
from jax.experimental import pallas as pl
from jax.experimental.pallas import tpu as pltpu
from jax.experimental.pallas import tpu_sc as plsc

N, E, D, H, C, DE = 10000, 320000, 128, 8, 128, 16
HC = H * C
NC, NS, L = 2, 16, 16
NW = NC * NS
EPW = E // NW
BB = 8
NBLK = EPW // BB
NPAD = 10112
NPT = NPAD // NS

SRCW = 1152
DSTW = 256
ACCW = 128
RECW = 128
RSQRT_C = 1.0 / float(C) ** 0.5

BN = 1000
GRID = N // BN


def _tables_body(x_r, a2f_r, wsm_r, csm_r, wv_r, bv_r,
                 src_o, dst_o, v_o, base0_o, res_o):
    xb = x_r[...]
    zk = jnp.dot(xb, a2f_r[...], preferred_element_type=jnp.float32)
    sm = jnp.dot(xb, wsm_r[...], preferred_element_type=jnp.float32) + csm_r[...]
    a1 = sm[:, 0:8]
    p = sm[:, 8:136]
    src_o[...] = jnp.concatenate(
        [zk, a1, jnp.zeros((xb.shape[0], SRCW - HC - 8), jnp.float32)], axis=1)
    dst_o[...] = jnp.concatenate([xb, p], axis=1)
    v_o[...] = jnp.dot(xb, wv_r[...], preferred_element_type=jnp.float32) + bv_r[...]
    base0_o[...] = sm[:, 136:264]
    res_o[...] = sm[:, 264:392]


def _tables(x, a2f, wsm, csm, wv, bv2):
    return pl.pallas_call(
        _tables_body,
        grid=(GRID,),
        in_specs=[
            pl.BlockSpec((BN, D), lambda i: (i, 0)),
            pl.BlockSpec((D, HC), lambda i: (0, 0)),
            pl.BlockSpec((D, 392), lambda i: (0, 0)),
            pl.BlockSpec((1, 392), lambda i: (0, 0)),
            pl.BlockSpec((D, HC), lambda i: (0, 0)),
            pl.BlockSpec((1, HC), lambda i: (0, 0)),
        ],
        out_specs=[
            pl.BlockSpec((BN, SRCW), lambda i: (i, 0)),
            pl.BlockSpec((BN, DSTW), lambda i: (i, 0)),
            pl.BlockSpec((BN, HC), lambda i: (i, 0)),
            pl.BlockSpec((BN, C), lambda i: (i, 0)),
            pl.BlockSpec((BN, C), lambda i: (i, 0)),
        ],
        out_shape=[
            jax.ShapeDtypeStruct((N, SRCW), jnp.float32),
            jax.ShapeDtypeStruct((N, DSTW), jnp.float32),
            jax.ShapeDtypeStruct((N, HC), jnp.float32),
            jax.ShapeDtypeStruct((N, C), jnp.float32),
            jax.ShapeDtypeStruct((N, C), jnp.float32),
        ],
    )(x, a2f, wsm, csm, wv, bv2)


def _pass1_body(src_hbm, dst_hbm, srcrow_hbm, dstrow_hbm, ea_hbm,
                ex_hbm, acc_hbm, accex_hbm,
                acc_sh, srcbuf, dstbuf, eabuf, sidx, didx, zstage, exstage,
                is0, is1, is2, is3, ea0, ea1, ea2, ea3,
                gs0, gs1, gd0, gd1, sc0, sc1, exw):
    cid = lax.axis_index("c")
    sid = lax.axis_index("s")
    wid = cid * NS + sid
    wbase = wid * EPW
    iota16 = lax.iota(jnp.int32, L)
    isems = (is0, is1, is2, is3)
    easems = (ea0, ea1, ea2, ea3)
    gss = (gs0, gs1)
    gds = (gd0, gd1)
    scs = (sc0, sc1)
    woff = pl.multiple_of(sid * NPT, 8)

    def zero_zslot(s):
        def zr(r, carry):
            for cc in range(ACCW // L):
                zstage[s, r, pl.ds(cc * L, L)] = jnp.zeros((L,), jnp.float32)
            return carry
        lax.fori_loop(0, BB, zr, 0)

    def zero_acc():
        for t in range(NPT // BB):
            off = pl.multiple_of(sid * NPT + t * BB, 8)
            pltpu.sync_copy(zstage.at[0], acc_sh.at[pl.ds(off, BB)])

    def base_of(g):
        return pl.multiple_of(wbase + g * BB, 8)

    def issue_idx(g, q):
        base = base_of(g)
        pltpu.async_copy(src_hbm.at[pl.ds(base, BB)], sidx.at[q], isems[q])
        pltpu.async_copy(dst_hbm.at[pl.ds(base, BB)], didx.at[q], isems[q])
        pltpu.async_copy(ea_hbm.at[pl.ds(base, BB)], eabuf.at[q], easems[q])

    def wait_idx(g, q):
        base = base_of(g)
        pltpu.make_async_copy(src_hbm.at[pl.ds(base, BB)], sidx.at[q],
                              isems[q]).wait()
        pltpu.make_async_copy(dst_hbm.at[pl.ds(base, BB)], didx.at[q],
                              isems[q]).wait()

    def issue_gather(c, q):
        pltpu.async_copy(srcrow_hbm.at[sidx.at[q]], srcbuf.at[c], gss[c])
        pltpu.async_copy(dstrow_hbm.at[didx.at[q]], dstbuf.at[c], gds[c])

    def wait_gather(c, q):
        pltpu.make_async_copy(srcrow_hbm.at[sidx.at[q]], srcbuf.at[c],
                              gss[c]).wait()
        pltpu.make_async_copy(dstrow_hbm.at[didx.at[q]], dstbuf.at[c],
                              gds[c]).wait()

    def wait_ea(g, q):
        base = base_of(g)
        pltpu.make_async_copy(ea_hbm.at[pl.ds(base, BB)], eabuf.at[q],
                              easems[q]).wait()

    def wait_scat(c, q):
        pltpu.make_async_copy(zstage.at[c], acc_sh.at[didx.at[q]],
                              scs[c]).wait()

    def wait_exw(g):
        pltpu.make_async_copy(exstage, ex_hbm.at[pl.ds(base_of(g), BB)],
                              exw).wait()

    def compute(c, q):
        def edge(e, carry):
            xv = [dstbuf[c, e, pl.ds(j * L, L)] for j in range(D // L)]
            eav = eabuf[q, e, pl.ds(0, L)]
            a1v = srcbuf[c, e, pl.ds(HC, L)]
            lvec = jnp.zeros((L,), jnp.float32)
            for h in range(H):
                acc = eav * dstbuf[c, e, pl.ds(D + h * L, L)]
                for j in range(D // L):
                    acc = acc + srcbuf[c, e, pl.ds(h * D + j * L, L)] * xv[j]
                for sh in (8, 4, 2, 1):
                    acc = acc + acc.at[iota16 ^ sh].get(mode="promise_in_bounds")
                lvec = jnp.where(iota16 == h, acc, lvec)
            lvec = (lvec + a1v) * RSQRT_C
            exv = jnp.exp(lvec)
            exv = jnp.where(iota16 < 8, exv, 0.0)
            exstage[e, pl.ds(0, L)] = exv
            for h in range(H):
                zstage[c, e, pl.ds(h * L, L)] = eav * exv[h]
            return carry
        lax.fori_loop(0, BB, edge, 0)

    def finish(g, c, q):
        pltpu.async_copy(zstage.at[c], acc_sh.at[didx.at[q]], scs[c], add=True)
        pltpu.async_copy(exstage, ex_hbm.at[pl.ds(base_of(g), BB)], exw)

    zero_zslot(0)
    zero_acc()
    plsc.subcore_barrier()

    issue_idx(0, 0)
    issue_idx(1, 1)
    wait_idx(0, 0)
    issue_gather(0, 0)
    wait_idx(1, 1)
    issue_gather(1, 1)
    wait_gather(0, 0)
    issue_idx(2, 2)
    wait_ea(0, 0)
    compute(0, 0)
    finish(0, 0, 0)
    wait_idx(2, 2)
    issue_gather(0, 2)
    wait_gather(1, 1)
    issue_idx(3, 3)
    wait_ea(1, 1)
    wait_exw(0)
    compute(1, 1)
    finish(1, 1, 1)

    def blk(G, carry):
        for j in range(4):
            g = G * 4 + j + 2
            c = j & 1
            q = (j + 2) & 3
            nq = (j + 3) & 3

            @pl.when(g + 1 < NBLK)
            def _():
                wait_idx(g + 1, nq)
                issue_gather(1 - c, nq)
            wait_gather(c, q)
            wait_scat(c, j)

            @pl.when(g + 2 < NBLK)
            def _():
                issue_idx(g + 2, j)
            wait_ea(g, q)
            wait_exw(g - 1)
            compute(c, q)
            finish(g, c, q)
        return carry

    lax.fori_loop(0, (NBLK - 2) // 4, blk, 0)
    wait_scat(0, 0)
    wait_scat(1, 1)
    wait_exw(NBLK - 1)
    plsc.subcore_barrier()
    pltpu.sync_copy(acc_sh.at[pl.ds(woff, NPT)],
                    acc_hbm.at[cid, pl.ds(woff, NPT)])

    zero_zslot(0)
    zero_zslot(1)
    zero_acc()
    plsc.subcore_barrier()

    def issue2(g, q):
        base = base_of(g)
        pltpu.async_copy(dst_hbm.at[pl.ds(base, BB)], didx.at[q], isems[q])
        pltpu.async_copy(ex_hbm.at[pl.ds(base, BB)], eabuf.at[q], easems[q])

    def wait2(g, q):
        base = base_of(g)
        pltpu.make_async_copy(dst_hbm.at[pl.ds(base, BB)], didx.at[q],
                              isems[q]).wait()
        pltpu.make_async_copy(ex_hbm.at[pl.ds(base, BB)], eabuf.at[q],
                              easems[q]).wait()

    def compute2(c, q):
        def edge(e, carry):
            zstage[c, e, pl.ds(0, L)] = eabuf[q, e, pl.ds(0, L)]
            return carry
        lax.fori_loop(0, BB, edge, 0)

    issue2(0, 0)
    issue2(1, 1)
    issue2(2, 2)
    wait2(0, 0)
    compute2(0, 0)
    pltpu.async_copy(zstage.at[0], acc_sh.at[didx.at[0]], sc0, add=True)
    issue2(3, 3)
    wait2(1, 1)
    compute2(1, 1)
    pltpu.async_copy(zstage.at[1], acc_sh.at[didx.at[1]], sc1, add=True)

    def blk2(G, carry):
        for j in range(4):
            g = G * 4 + j + 2
            c = j & 1
            q = (j + 2) & 3
            wait_scat(c, j)

            @pl.when(g + 2 < NBLK)
            def _():
                issue2(g + 2, j)
            wait2(g, q)
            compute2(c, q)
            pltpu.async_copy(zstage.at[c], acc_sh.at[didx.at[q]], scs[c],
                             add=True)
        return carry

    lax.fori_loop(0, (NBLK - 2) // 4, blk2, 0)
    wait_scat(0, 0)
    wait_scat(1, 1)
    plsc.subcore_barrier()
    pltpu.sync_copy(acc_sh.at[pl.ds(woff, NPT)],
                    accex_hbm.at[cid, pl.ds(woff, NPT)])


def _make_pass1():
    mesh = plsc.VectorSubcoreMesh(core_axis_name="c", subcore_axis_name="s",
                                  num_cores=NC, num_subcores=NS)
    return functools.partial(
        pl.kernel,
        out_type=[
            jax.ShapeDtypeStruct((E, L), jnp.float32),
            jax.ShapeDtypeStruct((NC, NPAD, ACCW), jnp.float32),
            jax.ShapeDtypeStruct((NC, NPAD, ACCW), jnp.float32),
        ],
        mesh=mesh,
        scratch_types=[
            pltpu.VMEM_SHARED((NPAD, ACCW), jnp.float32),
            pltpu.VMEM((2, BB, SRCW), jnp.float32),
            pltpu.VMEM((2, BB, DSTW), jnp.float32),
            pltpu.VMEM((4, BB, DE), jnp.float32),
            pltpu.VMEM((4, BB), jnp.int32),
            pltpu.VMEM((4, BB), jnp.int32),
            pltpu.VMEM((2, BB, ACCW), jnp.float32),
            pltpu.VMEM((BB, L), jnp.float32),
        ] + [pltpu.SemaphoreType.DMA] * 15,
    )(_pass1_body)


def _mid_body(acc_r, accex_r, base0_r, we2_r, rec_o, base_o):
    a = acc_r[0] + acc_r[1]
    seg = accex_r[0, :, 0:8] + accex_r[1, :, 0:8]
    rec = 1.0 / (seg * float(H) + 1e-30)
    rec_o[...] = jnp.concatenate(
        [rec, jnp.zeros((rec.shape[0], RECW - 8), jnp.float32)], axis=1)
    recb = jnp.reshape(jnp.broadcast_to(rec[:, :, None], (rec.shape[0], H, DE)),
                       (rec.shape[0], H * DE))
    zs = a * recb
    base_o[...] = base0_r[...] + jnp.dot(zs, we2_r[...],
                                         preferred_element_type=jnp.float32)


def _mid(acc, accex, base0, we2):
    return pl.pallas_call(
        _mid_body,
        grid=(GRID,),
        in_specs=[
            pl.BlockSpec((NC, BN, ACCW), lambda i: (0, i, 0)),
            pl.BlockSpec((NC, BN, ACCW), lambda i: (0, i, 0)),
            pl.BlockSpec((BN, C), lambda i: (i, 0)),
            pl.BlockSpec((H * DE, C), lambda i: (0, 0)),
        ],
        out_specs=[
            pl.BlockSpec((BN, RECW), lambda i: (i, 0)),
            pl.BlockSpec((BN, C), lambda i: (i, 0)),
        ],
        out_shape=[
            jax.ShapeDtypeStruct((N, RECW), jnp.float32),
            jax.ShapeDtypeStruct((N, C), jnp.float32),
        ],
    )(acc, accex, base0, we2)


def _pass2_body(src_hbm, dst_hbm, v_hbm, rec_hbm, exin_hbm, out_hbm,
                out_sh, vbuf, recbuf, exbuf, sidx, didx, wstage,
                is0, is1, is2, is3, ex0, ex1, ex2, ex3,
                gs0, gs1, gd0, gd1, sc0, sc1):
    cid = lax.axis_index("c")
    sid = lax.axis_index("s")
    wid = cid * NS + sid
    wbase = wid * EPW
    isems = (is0, is1, is2, is3)
    exsems = (ex0, ex1, ex2, ex3)
    gss = (gs0, gs1)
    gds = (gd0, gd1)
    scs = (sc0, sc1)
    woff = pl.multiple_of(sid * NPT, 8)

    def zero_wslot(s):
        def zr(r, carry):
            for cc in range(C // L):
                wstage[s, r, pl.ds(cc * L, L)] = jnp.zeros((L,), jnp.float32)
            return carry
        lax.fori_loop(0, BB, zr, 0)

    def base_of(g):
        return pl.multiple_of(wbase + g * BB, 8)

    def issue_idx(g, q):
        base = base_of(g)
        pltpu.async_copy(src_hbm.at[pl.ds(base, BB)], sidx.at[q], isems[q])
        pltpu.async_copy(dst_hbm.at[pl.ds(base, BB)], didx.at[q], isems[q])
        pltpu.async_copy(exin_hbm.at[pl.ds(base, BB)], exbuf.at[q], exsems[q])

    def wait_idx(g, q):
        base = base_of(g)
        pltpu.make_async_copy(src_hbm.at[pl.ds(base, BB)], sidx.at[q],
                              isems[q]).wait()
        pltpu.make_async_copy(dst_hbm.at[pl.ds(base, BB)], didx.at[q],
                              isems[q]).wait()

    def issue_gather(c, q):
        pltpu.async_copy(v_hbm.at[sidx.at[q]], vbuf.at[c], gss[c])
        pltpu.async_copy(rec_hbm.at[didx.at[q]], recbuf.at[c], gds[c])

    def wait_gather(c, q):
        pltpu.make_async_copy(v_hbm.at[sidx.at[q]], vbuf.at[c], gss[c]).wait()
        pltpu.make_async_copy(rec_hbm.at[didx.at[q]], recbuf.at[c],
                              gds[c]).wait()

    def wait_ex(g, q):
        base = base_of(g)
        pltpu.make_async_copy(exin_hbm.at[pl.ds(base, BB)], exbuf.at[q],
                              exsems[q]).wait()

    def wait_scat(c, q):
        pltpu.make_async_copy(wstage.at[c], out_sh.at[didx.at[q]],
                              scs[c]).wait()

    def compute(c, q):
        def edge(e, carry):
            av = exbuf[q, e, pl.ds(0, L)] * recbuf[c, e, pl.ds(0, L)]
            wv = [vbuf[c, e, pl.ds(j * L, L)] * av[0] for j in range(C // L)]
            for h in range(1, H):
                ah = av[h]
                for j in range(C // L):
                    wv[j] = wv[j] + vbuf[c, e, pl.ds(h * C + j * L, L)] * ah
            for j in range(C // L):
                wstage[c, e, pl.ds(j * L, L)] = wv[j]
            return carry
        lax.fori_loop(0, BB, edge, 0)

    zero_wslot(0)
    for t in range(NPT // BB):
        off = pl.multiple_of(sid * NPT + t * BB, 8)
        pltpu.sync_copy(wstage.at[0], out_sh.at[pl.ds(off, BB)])
    plsc.subcore_barrier()

    issue_idx(0, 0)
    issue_idx(1, 1)
    wait_idx(0, 0)
    issue_gather(0, 0)
    wait_idx(1, 1)
    issue_gather(1, 1)
    wait_gather(0, 0)
    issue_idx(2, 2)
    wait_ex(0, 0)
    compute(0, 0)
    pltpu.async_copy(wstage.at[0], out_sh.at[didx.at[0]], sc0, add=True)
    wait_idx(2, 2)
    issue_gather(0, 2)
    wait_gather(1, 1)
    issue_idx(3, 3)
    wait_ex(1, 1)
    compute(1, 1)
    pltpu.async_copy(wstage.at[1], out_sh.at[didx.at[1]], sc1, add=True)

    def blk(G, carry):
        for j in range(4):
            g = G * 4 + j + 2
            c = j & 1
            q = (j + 2) & 3
            nq = (j + 3) & 3

            @pl.when(g + 1 < NBLK)
            def _():
                wait_idx(g + 1, nq)
                issue_gather(1 - c, nq)
            wait_gather(c, q)
            wait_scat(c, j)

            @pl.when(g + 2 < NBLK)
            def _():
                issue_idx(g + 2, j)
            wait_ex(g, q)
            compute(c, q)
            pltpu.async_copy(wstage.at[c], out_sh.at[didx.at[q]], scs[c],
                             add=True)
        return carry

    lax.fori_loop(0, (NBLK - 2) // 4, blk, 0)
    wait_scat(0, 0)
    wait_scat(1, 1)
    plsc.subcore_barrier()
    pltpu.sync_copy(out_sh.at[pl.ds(woff, NPT)],
                    out_hbm.at[cid, pl.ds(woff, NPT)])


def _make_pass2():
    mesh = plsc.VectorSubcoreMesh(core_axis_name="c", subcore_axis_name="s",
                                  num_cores=NC, num_subcores=NS)
    return functools.partial(
        pl.kernel,
        out_type=jax.ShapeDtypeStruct((NC, NPAD, C), jnp.float32),
        mesh=mesh,
        scratch_types=[
            pltpu.VMEM_SHARED((NPAD, C), jnp.float32),
            pltpu.VMEM((2, BB, HC), jnp.float32),
            pltpu.VMEM((2, BB, RECW), jnp.float32),
            pltpu.VMEM((4, BB, L), jnp.float32),
            pltpu.VMEM((4, BB), jnp.int32),
            pltpu.VMEM((4, BB), jnp.int32),
            pltpu.VMEM((2, BB, C), jnp.float32),
        ] + [pltpu.SemaphoreType.DMA] * 14,
    )(_pass2_body)


def _final_body(ov_r, base_r, res_r, out_o):
    out_o[...] = jax.nn.relu(ov_r[0] + ov_r[1] + base_r[...]) + res_r[...]


def _final(outv, base, res):
    return pl.pallas_call(
        _final_body,
        grid=(GRID,),
        in_specs=[
            pl.BlockSpec((NC, BN, C), lambda i: (0, i, 0)),
            pl.BlockSpec((BN, C), lambda i: (i, 0)),
            pl.BlockSpec((BN, C), lambda i: (i, 0)),
        ],
        out_specs=pl.BlockSpec((BN, C), lambda i: (i, 0)),
        out_shape=jax.ShapeDtypeStruct((N, C), jnp.float32),
    )(outv, base, res)


def kernel(x, edge_index, edge_attr, Wq, bq, Wk, bk, Wv, bv, We, Wskip, bskip, Wres):
    src = edge_index[0]
    dst = edge_index[1]

    Wq3 = jnp.transpose(Wq.reshape(D, H, C), (1, 0, 2))
    Wk3 = jnp.transpose(Wk.reshape(D, H, C), (1, 0, 2))
    We3 = jnp.transpose(We.reshape(DE, H, C), (1, 0, 2))
    bq2 = bq.reshape(H, C)
    bk2 = bk.reshape(H, C)
    a2f = jnp.transpose(jnp.einsum("hdc,hec->hde", Wk3, Wq3), (1, 0, 2)) \
        .reshape(D, H * D)
    wp2 = jnp.transpose(jnp.einsum("hdc,hec->hde", Wq3, We3), (1, 0, 2)) \
        .reshape(D, H * DE)
    cp = jnp.einsum("hc,hec->he", bq2, We3).reshape(H * DE)
    wa1 = jnp.transpose((Wk3 * bq2[:, None, :]).sum(-1))
    ca1 = (bk2 * bq2).sum(-1)
    wsm = jnp.concatenate([wa1, wp2, Wskip, Wres], axis=1)
    csm = jnp.concatenate([ca1, cp, bskip,
                           jnp.zeros((C,), jnp.float32)]).reshape(1, 392)
    we2 = We3.reshape(H * DE, C)

    srcrow, dstrow, vtab, base0, res = _tables(x, a2f, wsm, csm, Wv,
                                               bv.reshape(1, HC))
    ex, acc, accex = _make_pass1()(src, dst, srcrow, dstrow, edge_attr)
    recrow, base = _mid(acc, accex, base0, we2)
    outv = _make_pass2()(src, dst, vtab, recrow, ex)
    return _final(outv, base, res)


# speedup vs baseline: 17.7831x; 1.0008x over previous
import functools

import jax
import jax.numpy as jnp
from jax import lax
from jax.experimental import pallas as pl
from jax.experimental.pallas import tpu as pltpu
from jax.experimental.pallas import tpu_sc as plsc

N, E, D, H, C, DE = 10000, 320000, 128, 8, 128, 16
HC = H * C
NC, NS, L = 2, 16, 16
NW = NC * NS
EPW = E // NW
BB = 8
NBLK = EPW // BB
NPAD = 10112
NPT = NPAD // NS

SRCW = 1152
DSTW = 256
ACCW = 128
RECW = 128
RSQRT_C = 1.0 / float(C) ** 0.5

BN = 1000
GRID = N // BN


def _tables_body(x_r, a2f_r, wsm_r, csm_r, wv_r, bv_r,
                 src_o, dst_o, v_o, base0_o, res_o):
    xb = x_r[...]
    zk = jnp.dot(xb, a2f_r[...], preferred_element_type=jnp.float32)
    sm = jnp.dot(xb, wsm_r[...], preferred_element_type=jnp.float32) + csm_r[...]
    a1 = sm[:, 0:8]
    p = sm[:, 8:136]
    src_o[...] = jnp.concatenate(
        [zk, a1, jnp.zeros((xb.shape[0], SRCW - HC - 8), jnp.float32)], axis=1)
    dst_o[...] = jnp.concatenate([xb, p], axis=1)
    v_o[...] = jnp.dot(xb, wv_r[...], preferred_element_type=jnp.float32) + bv_r[...]
    base0_o[...] = sm[:, 136:264]
    res_o[...] = sm[:, 264:392]


def _tables(x, a2f, wsm, csm, wv, bv2):
    return pl.pallas_call(
        _tables_body,
        grid=(GRID,),
        in_specs=[
            pl.BlockSpec((BN, D), lambda i: (i, 0)),
            pl.BlockSpec((D, HC), lambda i: (0, 0)),
            pl.BlockSpec((D, 392), lambda i: (0, 0)),
            pl.BlockSpec((1, 392), lambda i: (0, 0)),
            pl.BlockSpec((D, HC), lambda i: (0, 0)),
            pl.BlockSpec((1, HC), lambda i: (0, 0)),
        ],
        out_specs=[
            pl.BlockSpec((BN, SRCW), lambda i: (i, 0)),
            pl.BlockSpec((BN, DSTW), lambda i: (i, 0)),
            pl.BlockSpec((BN, HC), lambda i: (i, 0)),
            pl.BlockSpec((BN, C), lambda i: (i, 0)),
            pl.BlockSpec((BN, C), lambda i: (i, 0)),
        ],
        out_shape=[
            jax.ShapeDtypeStruct((N, SRCW), jnp.float32),
            jax.ShapeDtypeStruct((N, DSTW), jnp.float32),
            jax.ShapeDtypeStruct((N, HC), jnp.float32),
            jax.ShapeDtypeStruct((N, C), jnp.float32),
            jax.ShapeDtypeStruct((N, C), jnp.float32),
        ],
    )(x, a2f, wsm, csm, wv, bv2)


def _pass1_body(src_hbm, dst_hbm, srcrow_hbm, dstrow_hbm, ea_hbm,
                ex_hbm, acc_hbm, accex_hbm,
                acc_sh, srcbuf, dstbuf, eabuf, sidx, didx, zstage, exstage,
                is0, is1, is2, is3, ea0, ea1, ea2, ea3,
                gs0, gs1, gd0, gd1, sc0, sc1, exw):
    cid = lax.axis_index("c")
    sid = lax.axis_index("s")
    wid = cid * NS + sid
    wbase = wid * EPW
    iota16 = lax.iota(jnp.int32, L)
    isems = (is0, is1, is2, is3)
    easems = (ea0, ea1, ea2, ea3)
    gss = (gs0, gs1)
    gds = (gd0, gd1)
    scs = (sc0, sc1)
    woff = pl.multiple_of(sid * NPT, 8)

    def zero_zslot(s):
        def zr(r, carry):
            for cc in range(ACCW // L):
                zstage[s, r, pl.ds(cc * L, L)] = jnp.zeros((L,), jnp.float32)
            return carry
        lax.fori_loop(0, BB, zr, 0)

    def zero_acc():
        for t in range(NPT // BB):
            off = pl.multiple_of(sid * NPT + t * BB, 8)
            pltpu.sync_copy(zstage.at[0], acc_sh.at[pl.ds(off, BB)])

    def base_of(g):
        return pl.multiple_of(wbase + g * BB, 8)

    def issue_idx(g, q):
        base = base_of(g)
        pltpu.async_copy(src_hbm.at[pl.ds(base, BB)], sidx.at[q], isems[q])
        pltpu.async_copy(dst_hbm.at[pl.ds(base, BB)], didx.at[q], isems[q])
        pltpu.async_copy(ea_hbm.at[pl.ds(base, BB)], eabuf.at[q], easems[q])

    def wait_idx(g, q):
        base = base_of(g)
        pltpu.make_async_copy(src_hbm.at[pl.ds(base, BB)], sidx.at[q],
                              isems[q]).wait()
        pltpu.make_async_copy(dst_hbm.at[pl.ds(base, BB)], didx.at[q],
                              isems[q]).wait()

    def issue_gather(c, q):
        pltpu.async_copy(srcrow_hbm.at[sidx.at[q]], srcbuf.at[c], gss[c])
        pltpu.async_copy(dstrow_hbm.at[didx.at[q]], dstbuf.at[c], gds[c])

    def wait_gather(c, q):
        pltpu.make_async_copy(srcrow_hbm.at[sidx.at[q]], srcbuf.at[c],
                              gss[c]).wait()
        pltpu.make_async_copy(dstrow_hbm.at[didx.at[q]], dstbuf.at[c],
                              gds[c]).wait()

    def wait_ea(g, q):
        base = base_of(g)
        pltpu.make_async_copy(ea_hbm.at[pl.ds(base, BB)], eabuf.at[q],
                              easems[q]).wait()

    def wait_scat(c, q):
        pltpu.make_async_copy(zstage.at[c], acc_sh.at[didx.at[q]],
                              scs[c]).wait()

    def wait_exw(g):
        pltpu.make_async_copy(exstage, ex_hbm.at[pl.ds(base_of(g), BB)],
                              exw).wait()

    def compute(c, q):
        def edge(e, carry):
            xv = [dstbuf[c, e, pl.ds(j * L, L)] for j in range(D // L)]
            eav = eabuf[q, e, pl.ds(0, L)]
            a1v = srcbuf[c, e, pl.ds(HC, L)]
            accs = []
            for h in range(H):
                acc = eav * dstbuf[c, e, pl.ds(D + h * L, L)]
                for j in range(D // L):
                    acc = acc + srcbuf[c, e, pl.ds(h * D + j * L, L)] * xv[j]
                accs.append(acc + acc.at[iota16 ^ 8].get(mode="promise_in_bounds"))
            for d in (4, 2, 1):
                nxt = []
                for j in range(len(accs) // 2):
                    a = accs[j]
                    b = accs[j + len(accs) // 2]
                    ta = a + a.at[iota16 ^ d].get(mode="promise_in_bounds")
                    tb = b + b.at[iota16 ^ d].get(mode="promise_in_bounds")
                    nxt.append(jnp.where((iota16 & d) == 0, ta, tb))
                accs = nxt
            lvec = (accs[0] + a1v) * RSQRT_C
            exv = jnp.exp(lvec)
            exv = jnp.where(iota16 < 8, exv, 0.0)
            exstage[e, pl.ds(0, L)] = exv
            for h in range(H):
                zstage[c, e, pl.ds(h * L, L)] = eav * exv[h]
            return carry
        lax.fori_loop(0, BB, edge, 0)

    def finish(g, c, q):
        pltpu.async_copy(zstage.at[c], acc_sh.at[didx.at[q]], scs[c], add=True)
        pltpu.async_copy(exstage, ex_hbm.at[pl.ds(base_of(g), BB)], exw)

    zero_zslot(0)
    zero_acc()
    plsc.subcore_barrier()

    issue_idx(0, 0)
    issue_idx(1, 1)
    wait_idx(0, 0)
    issue_gather(0, 0)
    wait_idx(1, 1)
    issue_gather(1, 1)
    wait_gather(0, 0)
    issue_idx(2, 2)
    wait_ea(0, 0)
    compute(0, 0)
    finish(0, 0, 0)
    wait_idx(2, 2)
    issue_gather(0, 2)
    wait_gather(1, 1)
    issue_idx(3, 3)
    wait_ea(1, 1)
    wait_exw(0)
    compute(1, 1)
    finish(1, 1, 1)

    def blk(G, carry):
        for j in range(4):
            g = G * 4 + j + 2
            c = j & 1
            q = (j + 2) & 3
            nq = (j + 3) & 3

            @pl.when(g + 1 < NBLK)
            def _():
                wait_idx(g + 1, nq)
                issue_gather(1 - c, nq)
            wait_gather(c, q)
            wait_scat(c, j)

            @pl.when(g + 2 < NBLK)
            def _():
                issue_idx(g + 2, j)
            wait_ea(g, q)
            wait_exw(g - 1)
            compute(c, q)
            finish(g, c, q)
        return carry

    lax.fori_loop(0, (NBLK - 2) // 4, blk, 0)
    wait_scat(0, 0)
    wait_scat(1, 1)
    wait_exw(NBLK - 1)
    plsc.subcore_barrier()
    pltpu.sync_copy(acc_sh.at[pl.ds(woff, NPT)],
                    acc_hbm.at[cid, pl.ds(woff, NPT)])

    zero_zslot(0)
    zero_zslot(1)
    zero_acc()
    plsc.subcore_barrier()

    def issue2(g, q):
        base = base_of(g)
        pltpu.async_copy(dst_hbm.at[pl.ds(base, BB)], didx.at[q], isems[q])
        pltpu.async_copy(ex_hbm.at[pl.ds(base, BB)], eabuf.at[q], easems[q])

    def wait2(g, q):
        base = base_of(g)
        pltpu.make_async_copy(dst_hbm.at[pl.ds(base, BB)], didx.at[q],
                              isems[q]).wait()
        pltpu.make_async_copy(ex_hbm.at[pl.ds(base, BB)], eabuf.at[q],
                              easems[q]).wait()

    def compute2(c, q):
        def edge(e, carry):
            zstage[c, e, pl.ds(0, L)] = eabuf[q, e, pl.ds(0, L)]
            return carry
        lax.fori_loop(0, BB, edge, 0)

    issue2(0, 0)
    issue2(1, 1)
    issue2(2, 2)
    wait2(0, 0)
    compute2(0, 0)
    pltpu.async_copy(zstage.at[0], acc_sh.at[didx.at[0]], sc0, add=True)
    issue2(3, 3)
    wait2(1, 1)
    compute2(1, 1)
    pltpu.async_copy(zstage.at[1], acc_sh.at[didx.at[1]], sc1, add=True)

    def blk2(G, carry):
        for j in range(4):
            g = G * 4 + j + 2
            c = j & 1
            q = (j + 2) & 3
            wait_scat(c, j)

            @pl.when(g + 2 < NBLK)
            def _():
                issue2(g + 2, j)
            wait2(g, q)
            compute2(c, q)
            pltpu.async_copy(zstage.at[c], acc_sh.at[didx.at[q]], scs[c],
                             add=True)
        return carry

    lax.fori_loop(0, (NBLK - 2) // 4, blk2, 0)
    wait_scat(0, 0)
    wait_scat(1, 1)
    plsc.subcore_barrier()
    pltpu.sync_copy(acc_sh.at[pl.ds(woff, NPT)],
                    accex_hbm.at[cid, pl.ds(woff, NPT)])


def _make_pass1():
    mesh = plsc.VectorSubcoreMesh(core_axis_name="c", subcore_axis_name="s",
                                  num_cores=NC, num_subcores=NS)
    return functools.partial(
        pl.kernel,
        out_type=[
            jax.ShapeDtypeStruct((E, L), jnp.float32),
            jax.ShapeDtypeStruct((NC, NPAD, ACCW), jnp.float32),
            jax.ShapeDtypeStruct((NC, NPAD, ACCW), jnp.float32),
        ],
        mesh=mesh,
        scratch_types=[
            pltpu.VMEM_SHARED((NPAD, ACCW), jnp.float32),
            pltpu.VMEM((2, BB, SRCW), jnp.float32),
            pltpu.VMEM((2, BB, DSTW), jnp.float32),
            pltpu.VMEM((4, BB, DE), jnp.float32),
            pltpu.VMEM((4, BB), jnp.int32),
            pltpu.VMEM((4, BB), jnp.int32),
            pltpu.VMEM((2, BB, ACCW), jnp.float32),
            pltpu.VMEM((BB, L), jnp.float32),
        ] + [pltpu.SemaphoreType.DMA] * 15,
    )(_pass1_body)


def _mid_body(acc_r, accex_r, base0_r, we2_r, rec_o, base_o):
    a = acc_r[0] + acc_r[1]
    seg = accex_r[0, :, 0:8] + accex_r[1, :, 0:8]
    rec = 1.0 / (seg * float(H) + 1e-30)
    rec_o[...] = jnp.concatenate(
        [rec, jnp.zeros((rec.shape[0], RECW - 8), jnp.float32)], axis=1)
    recb = jnp.reshape(jnp.broadcast_to(rec[:, :, None], (rec.shape[0], H, DE)),
                       (rec.shape[0], H * DE))
    zs = a * recb
    base_o[...] = base0_r[...] + jnp.dot(zs, we2_r[...],
                                         preferred_element_type=jnp.float32)


def _mid(acc, accex, base0, we2):
    return pl.pallas_call(
        _mid_body,
        grid=(GRID,),
        in_specs=[
            pl.BlockSpec((NC, BN, ACCW), lambda i: (0, i, 0)),
            pl.BlockSpec((NC, BN, ACCW), lambda i: (0, i, 0)),
            pl.BlockSpec((BN, C), lambda i: (i, 0)),
            pl.BlockSpec((H * DE, C), lambda i: (0, 0)),
        ],
        out_specs=[
            pl.BlockSpec((BN, RECW), lambda i: (i, 0)),
            pl.BlockSpec((BN, C), lambda i: (i, 0)),
        ],
        out_shape=[
            jax.ShapeDtypeStruct((N, RECW), jnp.float32),
            jax.ShapeDtypeStruct((N, C), jnp.float32),
        ],
    )(acc, accex, base0, we2)


def _pass2_body(src_hbm, dst_hbm, v_hbm, rec_hbm, exin_hbm, out_hbm,
                out_sh, vbuf, recbuf, exbuf, sidx, didx, wstage,
                is0, is1, is2, is3, ex0, ex1, ex2, ex3,
                gs0, gs1, gd0, gd1, sc0, sc1):
    cid = lax.axis_index("c")
    sid = lax.axis_index("s")
    wid = cid * NS + sid
    wbase = wid * EPW
    isems = (is0, is1, is2, is3)
    exsems = (ex0, ex1, ex2, ex3)
    gss = (gs0, gs1)
    gds = (gd0, gd1)
    scs = (sc0, sc1)
    woff = pl.multiple_of(sid * NPT, 8)

    def zero_wslot(s):
        def zr(r, carry):
            for cc in range(C // L):
                wstage[s, r, pl.ds(cc * L, L)] = jnp.zeros((L,), jnp.float32)
            return carry
        lax.fori_loop(0, BB, zr, 0)

    def base_of(g):
        return pl.multiple_of(wbase + g * BB, 8)

    def issue_idx(g, q):
        base = base_of(g)
        pltpu.async_copy(src_hbm.at[pl.ds(base, BB)], sidx.at[q], isems[q])
        pltpu.async_copy(dst_hbm.at[pl.ds(base, BB)], didx.at[q], isems[q])
        pltpu.async_copy(exin_hbm.at[pl.ds(base, BB)], exbuf.at[q], exsems[q])

    def wait_idx(g, q):
        base = base_of(g)
        pltpu.make_async_copy(src_hbm.at[pl.ds(base, BB)], sidx.at[q],
                              isems[q]).wait()
        pltpu.make_async_copy(dst_hbm.at[pl.ds(base, BB)], didx.at[q],
                              isems[q]).wait()

    def issue_gather(c, q):
        pltpu.async_copy(v_hbm.at[sidx.at[q]], vbuf.at[c], gss[c])
        pltpu.async_copy(rec_hbm.at[didx.at[q]], recbuf.at[c], gds[c])

    def wait_gather(c, q):
        pltpu.make_async_copy(v_hbm.at[sidx.at[q]], vbuf.at[c], gss[c]).wait()
        pltpu.make_async_copy(rec_hbm.at[didx.at[q]], recbuf.at[c],
                              gds[c]).wait()

    def wait_ex(g, q):
        base = base_of(g)
        pltpu.make_async_copy(exin_hbm.at[pl.ds(base, BB)], exbuf.at[q],
                              exsems[q]).wait()

    def wait_scat(c, q):
        pltpu.make_async_copy(wstage.at[c], out_sh.at[didx.at[q]],
                              scs[c]).wait()

    def compute(c, q):
        def edge(e, carry):
            av = exbuf[q, e, pl.ds(0, L)] * recbuf[c, e, pl.ds(0, L)]
            wv = [vbuf[c, e, pl.ds(j * L, L)] * av[0] for j in range(C // L)]
            for h in range(1, H):
                ah = av[h]
                for j in range(C // L):
                    wv[j] = wv[j] + vbuf[c, e, pl.ds(h * C + j * L, L)] * ah
            for j in range(C // L):
                wstage[c, e, pl.ds(j * L, L)] = wv[j]
            return carry
        lax.fori_loop(0, BB, edge, 0)

    zero_wslot(0)
    for t in range(NPT // BB):
        off = pl.multiple_of(sid * NPT + t * BB, 8)
        pltpu.sync_copy(wstage.at[0], out_sh.at[pl.ds(off, BB)])
    plsc.subcore_barrier()

    issue_idx(0, 0)
    issue_idx(1, 1)
    wait_idx(0, 0)
    issue_gather(0, 0)
    wait_idx(1, 1)
    issue_gather(1, 1)
    wait_gather(0, 0)
    issue_idx(2, 2)
    wait_ex(0, 0)
    compute(0, 0)
    pltpu.async_copy(wstage.at[0], out_sh.at[didx.at[0]], sc0, add=True)
    wait_idx(2, 2)
    issue_gather(0, 2)
    wait_gather(1, 1)
    issue_idx(3, 3)
    wait_ex(1, 1)
    compute(1, 1)
    pltpu.async_copy(wstage.at[1], out_sh.at[didx.at[1]], sc1, add=True)

    def blk(G, carry):
        for j in range(4):
            g = G * 4 + j + 2
            c = j & 1
            q = (j + 2) & 3
            nq = (j + 3) & 3

            @pl.when(g + 1 < NBLK)
            def _():
                wait_idx(g + 1, nq)
                issue_gather(1 - c, nq)
            wait_gather(c, q)
            wait_scat(c, j)

            @pl.when(g + 2 < NBLK)
            def _():
                issue_idx(g + 2, j)
            wait_ex(g, q)
            compute(c, q)
            pltpu.async_copy(wstage.at[c], out_sh.at[didx.at[q]], scs[c],
                             add=True)
        return carry

    lax.fori_loop(0, (NBLK - 2) // 4, blk, 0)
    wait_scat(0, 0)
    wait_scat(1, 1)
    plsc.subcore_barrier()
    pltpu.sync_copy(out_sh.at[pl.ds(woff, NPT)],
                    out_hbm.at[cid, pl.ds(woff, NPT)])


def _make_pass2():
    mesh = plsc.VectorSubcoreMesh(core_axis_name="c", subcore_axis_name="s",
                                  num_cores=NC, num_subcores=NS)
    return functools.partial(
        pl.kernel,
        out_type=jax.ShapeDtypeStruct((NC, NPAD, C), jnp.float32),
        mesh=mesh,
        scratch_types=[
            pltpu.VMEM_SHARED((NPAD, C), jnp.float32),
            pltpu.VMEM((2, BB, HC), jnp.float32),
            pltpu.VMEM((2, BB, RECW), jnp.float32),
            pltpu.VMEM((4, BB, L), jnp.float32),
            pltpu.VMEM((4, BB), jnp.int32),
            pltpu.VMEM((4, BB), jnp.int32),
            pltpu.VMEM((2, BB, C), jnp.float32),
        ] + [pltpu.SemaphoreType.DMA] * 14,
    )(_pass2_body)


def _final_body(ov_r, base_r, res_r, out_o):
    out_o[...] = jax.nn.relu(ov_r[0] + ov_r[1] + base_r[...]) + res_r[...]


def _final(outv, base, res):
    return pl.pallas_call(
        _final_body,
        grid=(GRID,),
        in_specs=[
            pl.BlockSpec((NC, BN, C), lambda i: (0, i, 0)),
            pl.BlockSpec((BN, C), lambda i: (i, 0)),
            pl.BlockSpec((BN, C), lambda i: (i, 0)),
        ],
        out_specs=pl.BlockSpec((BN, C), lambda i: (i, 0)),
        out_shape=jax.ShapeDtypeStruct((N, C), jnp.float32),
    )(outv, base, res)


def kernel(x, edge_index, edge_attr, Wq, bq, Wk, bk, Wv, bv, We, Wskip, bskip, Wres):
    src = edge_index[0]
    dst = edge_index[1]

    Wq3 = jnp.transpose(Wq.reshape(D, H, C), (1, 0, 2))
    Wk3 = jnp.transpose(Wk.reshape(D, H, C), (1, 0, 2))
    We3 = jnp.transpose(We.reshape(DE, H, C), (1, 0, 2))
    bq2 = bq.reshape(H, C)
    bk2 = bk.reshape(H, C)
    a2f = jnp.transpose(jnp.einsum("hdc,hec->hde", Wk3, Wq3), (1, 0, 2)) \
        .reshape(D, H * D)
    wp2 = jnp.transpose(jnp.einsum("hdc,hec->hde", Wq3, We3), (1, 0, 2)) \
        .reshape(D, H * DE)
    cp = jnp.einsum("hc,hec->he", bq2, We3).reshape(H * DE)
    wa1 = jnp.transpose((Wk3 * bq2[:, None, :]).sum(-1))
    ca1 = (bk2 * bq2).sum(-1)
    wsm = jnp.concatenate([wa1, wp2, Wskip, Wres], axis=1)
    csm = jnp.concatenate([ca1, cp, bskip,
                           jnp.zeros((C,), jnp.float32)]).reshape(1, 392)
    we2 = We3.reshape(H * DE, C)

    srcrow, dstrow, vtab, base0, res = _tables(x, a2f, wsm, csm, Wv,
                                               bv.reshape(1, HC))
    ex, acc, accex = _make_pass1()(src, dst, srcrow, dstrow, edge_attr)
    recrow, base = _mid(acc, accex, base0, we2)
    outv = _make_pass2()(src, dst, vtab, recrow, ex)
    return _final(outv, base, res)
